# Initial kernel scaffold; baseline (speedup 1.0000x reference)
import jax
import jax.numpy as jnp
from jax.experimental import pallas as pl


def kernel(edge_index, x):
    raise NotImplementedError("write your pallas kernel here")


# speedup vs baseline: 5.1099x; 5.1099x over previous
import functools

import jax
import jax.numpy as jnp
from jax import lax
from jax.experimental import pallas as pl
from jax.experimental.pallas import tpu as pltpu
from jax.experimental.pallas import tpu_sc as plsc

N_NODES = 10000
D_FEAT = 256
N_EDGES = 160000

N_SUBCORES = 16
HALF = D_FEAT // 2
EDGES_PER_TILE = N_EDGES // N_SUBCORES
CHUNK = 80
N_CHUNKS = EDGES_PER_TILE // CHUNK
ROWS_PER_TILE = N_NODES // N_SUBCORES
ZROWS = 125


def _sc_body(x_hbm, src_hbm, dst_hbm, out_hbm,
             src_v, dst_v, rows_v, acc_sh, sem):
    c = lax.axis_index("c")
    s = lax.axis_index("s")

    pltpu.sync_copy(src_hbm.at[s], src_v)
    pltpu.sync_copy(dst_hbm.at[s], dst_v)

    def tr_row(j, _):
        def tr_col(k, _):
            v = src_v[j, pl.ds(k * 16, 16)]
            src_v[j, pl.ds(k * 16, 16)] = v * 2 + c
            return 0
        return lax.fori_loop(0, CHUNK // 16, tr_col, 0)
    lax.fori_loop(0, N_CHUNKS, tr_row, 0)

    def z_row(i, _):
        def z_col(k, _):
            rows_v[i, pl.ds(k * 16, 16)] = jnp.zeros((16,), jnp.float32)
            return 0
        return lax.fori_loop(0, HALF // 16, z_col, 0)
    lax.fori_loop(0, CHUNK, z_row, 0)
    base = s * ROWS_PER_TILE
    for t in range(ROWS_PER_TILE // CHUNK):
        pltpu.sync_copy(rows_v, acc_sh.at[pl.ds(base + t * CHUNK, CHUNK)])
    rem = ROWS_PER_TILE % CHUNK
    if rem:
        pltpu.sync_copy(rows_v.at[pl.ds(0, rem)],
                        acc_sh.at[pl.ds(base + (ROWS_PER_TILE // CHUNK) * CHUNK, rem)])
    plsc.subcore_barrier()

    def step(j, _):
        pltpu.async_copy(x_hbm.at[src_v.at[j]], rows_v, sem).wait()
        pltpu.sync_copy(rows_v, acc_sh.at[dst_v.at[j]], add=True)
        return 0
    lax.fori_loop(0, N_CHUNKS, step, 0)
    plsc.subcore_barrier()

    pltpu.sync_copy(acc_sh.at[pl.ds(s * ROWS_PER_TILE, ROWS_PER_TILE)],
                    out_hbm.at[pl.ds(s * ROWS_PER_TILE, ROWS_PER_TILE), c])


@jax.jit
def _message_passing(x2, src, dst):
    mesh = plsc.VectorSubcoreMesh(core_axis_name="c", subcore_axis_name="s")
    fn = functools.partial(
        pl.kernel,
        mesh=mesh,
        out_type=jax.ShapeDtypeStruct((N_NODES, 2, HALF), jnp.float32),
        scratch_types=[
            pltpu.VMEM((N_CHUNKS, CHUNK), jnp.int32),
            pltpu.VMEM((N_CHUNKS, CHUNK), jnp.int32),
            pltpu.VMEM((CHUNK, HALF), jnp.float32),
            pltpu.VMEM_SHARED((N_NODES, HALF), jnp.float32),
            pltpu.SemaphoreType.DMA,
        ],
    )(_sc_body)
    return fn(x2, src, dst)


def kernel(edge_index, x):
    ei = edge_index.astype(jnp.int32)
    src = ei[0].reshape(N_SUBCORES, N_CHUNKS, CHUNK)
    dst = ei[1].reshape(N_SUBCORES, N_CHUNKS, CHUNK)
    x2 = x.reshape(2 * N_NODES, HALF)
    out = _message_passing(x2, src, dst)
    return out.reshape(N_NODES, D_FEAT)


# speedup vs baseline: 7.8309x; 1.5325x over previous
import functools

import jax
import jax.numpy as jnp
from jax import lax
from jax.experimental import pallas as pl
from jax.experimental.pallas import tpu as pltpu
from jax.experimental.pallas import tpu_sc as plsc

N_NODES = 10000
D_FEAT = 256
N_EDGES = 160000

N_SUBCORES = 16
HALF = D_FEAT // 2
EDGES_PER_TILE = N_EDGES // N_SUBCORES
CHUNK = 80
N_CHUNKS = EDGES_PER_TILE // CHUNK
ROWS_PER_TILE = N_NODES // N_SUBCORES


def _sc_body(x_hbm, src_hbm, dst_hbm, out_hbm,
             src_v, dst_v, rows_a, rows_b, acc_sh, sem0, sem1):
    c = lax.axis_index("c")
    s = lax.axis_index("s")
    sems = (sem0, sem1)
    bufs = (rows_a, rows_b)

    pltpu.sync_copy(src_hbm.at[c, s], src_v)
    pltpu.sync_copy(dst_hbm.at[s], dst_v)

    def z_row(i, _):
        def z_col(k, _):
            rows_a[i, pl.ds(k * 16, 16)] = jnp.zeros((16,), jnp.float32)
            return 0
        return lax.fori_loop(0, HALF // 16, z_col, 0)
    lax.fori_loop(0, CHUNK, z_row, 0)
    base = s * ROWS_PER_TILE
    for t in range(ROWS_PER_TILE // CHUNK):
        pltpu.sync_copy(rows_a, acc_sh.at[pl.ds(base + t * CHUNK, CHUNK)])
    rem = ROWS_PER_TILE % CHUNK
    if rem:
        pltpu.sync_copy(rows_a.at[pl.ds(0, rem)],
                        acc_sh.at[pl.ds(base + (ROWS_PER_TILE // CHUNK) * CHUNK, rem)])
    plsc.subcore_barrier()

    def start(j, b):
        pltpu.async_copy(x_hbm.at[src_v.at[pl.ds(j * CHUNK, CHUNK)]],
                         bufs[b], sems[b])

    def finish(j, b):
        pltpu.make_async_copy(x_hbm.at[src_v.at[pl.ds(j * CHUNK, CHUNK)]],
                              bufs[b], sems[b]).wait()
        pltpu.sync_copy(bufs[b], acc_sh.at[dst_v.at[j]], add=True)

    start(0, 0)

    def pair(i, _):
        j = 2 * i
        start(j + 1, 1)
        finish(j, 0)
        start(j + 2, 0)
        finish(j + 1, 1)
        return 0
    lax.fori_loop(0, (N_CHUNKS - 1) // 2, pair, 0)
    finish(N_CHUNKS - 1, (N_CHUNKS - 1) % 2)
    plsc.subcore_barrier()

    pltpu.sync_copy(acc_sh.at[pl.ds(s * ROWS_PER_TILE, ROWS_PER_TILE)],
                    out_hbm.at[pl.ds(s * ROWS_PER_TILE, ROWS_PER_TILE), c])


@jax.jit
def _message_passing(x2, src2, dst):
    mesh = plsc.VectorSubcoreMesh(core_axis_name="c", subcore_axis_name="s")
    fn = functools.partial(
        pl.kernel,
        mesh=mesh,
        out_type=jax.ShapeDtypeStruct((N_NODES, 2, HALF), jnp.float32),
        scratch_types=[
            pltpu.VMEM((EDGES_PER_TILE,), jnp.int32),
            pltpu.VMEM((N_CHUNKS, CHUNK), jnp.int32),
            pltpu.VMEM((CHUNK, HALF), jnp.float32),
            pltpu.VMEM((CHUNK, HALF), jnp.float32),
            pltpu.VMEM_SHARED((N_NODES, HALF), jnp.float32),
            pltpu.SemaphoreType.DMA,
            pltpu.SemaphoreType.DMA,
        ],
    )(_sc_body)
    return fn(x2, src2, dst)


def kernel(edge_index, x):
    ei = edge_index.astype(jnp.int32)
    src = ei[0].reshape(1, N_SUBCORES, EDGES_PER_TILE)
    src2 = 2 * jnp.concatenate([src, src], axis=0) + jnp.array([0, 1], jnp.int32).reshape(2, 1, 1)
    dst = ei[1].reshape(N_SUBCORES, N_CHUNKS, CHUNK)
    x2 = x.reshape(2 * N_NODES, HALF)
    out = _message_passing(x2, src2, dst)
    return out.reshape(N_NODES, D_FEAT)


# speedup vs baseline: 7.8718x; 1.0052x over previous
import functools

import jax
import jax.numpy as jnp
from jax import lax
from jax.experimental import pallas as pl
from jax.experimental.pallas import tpu as pltpu
from jax.experimental.pallas import tpu_sc as plsc

N_NODES = 10000
D_FEAT = 256
N_EDGES = 160000

N_SUBCORES = 16
HALF = D_FEAT // 2
EDGES_PER_TILE = N_EDGES // N_SUBCORES
CHUNK = 80
N_CHUNKS = EDGES_PER_TILE // CHUNK
ROWS_PER_TILE = N_NODES // N_SUBCORES


def _sc_body(x_hbm, src_hbm, dst_hbm, out_hbm,
             src_v, dst_v, rows_a, rows_b, acc_sh,
             gsem0, gsem1, ssem0, ssem1):
    c = lax.axis_index("c")
    s = lax.axis_index("s")
    gsems = (gsem0, gsem1)
    ssems = (ssem0, ssem1)
    bufs = (rows_a, rows_b)

    def gstart(j, b):
        pltpu.async_copy(x_hbm.at[src_v.at[pl.ds(j * CHUNK, CHUNK)]],
                         bufs[b], gsems[b])

    def gwait(j, b):
        pltpu.make_async_copy(x_hbm.at[src_v.at[pl.ds(j * CHUNK, CHUNK)]],
                              bufs[b], gsems[b]).wait()

    def sstart(j, b):
        pltpu.async_copy(bufs[b], acc_sh.at[dst_v.at[j]], ssems[b], add=True)

    def swait(j, b):
        pltpu.make_async_copy(bufs[b], acc_sh.at[dst_v.at[j]], ssems[b]).wait()

    pltpu.sync_copy(src_hbm.at[c, s], src_v)
    pltpu.sync_copy(dst_hbm.at[s], dst_v)
    gstart(0, 0)

    def z_row(i, _):
        def z_col(k, _):
            rows_b[i, pl.ds(k * 16, 16)] = jnp.zeros((16,), jnp.float32)
            return 0
        return lax.fori_loop(0, HALF // 16, z_col, 0)
    lax.fori_loop(0, CHUNK, z_row, 0)
    base = s * ROWS_PER_TILE
    for t in range(ROWS_PER_TILE // CHUNK):
        pltpu.sync_copy(rows_b, acc_sh.at[pl.ds(base + t * CHUNK, CHUNK)])
    rem = ROWS_PER_TILE % CHUNK
    if rem:
        pltpu.sync_copy(rows_b.at[pl.ds(0, rem)],
                        acc_sh.at[pl.ds(base + (ROWS_PER_TILE // CHUNK) * CHUNK, rem)])
    plsc.subcore_barrier()

    gstart(1, 1)
    gwait(0, 0)
    sstart(0, 0)

    def pair(i, _):
        j = 2 * i + 1
        swait(j - 1, 0)
        gstart(j + 1, 0)
        gwait(j, 1)
        sstart(j, 1)
        swait(j, 1)
        gstart(j + 2, 1)
        gwait(j + 1, 0)
        sstart(j + 1, 0)
        return 0
    lax.fori_loop(0, (N_CHUNKS - 3) // 2, pair, 0)

    j_last = N_CHUNKS - 1
    swait(j_last - 2, 0)
    gstart(j_last, 0)
    gwait(j_last - 1, 1)
    sstart(j_last - 1, 1)
    swait(j_last - 1, 1)
    gwait(j_last, 0)
    sstart(j_last, 0)
    swait(j_last, 0)
    plsc.subcore_barrier()

    pltpu.sync_copy(acc_sh.at[pl.ds(s * ROWS_PER_TILE, ROWS_PER_TILE)],
                    out_hbm.at[pl.ds(s * ROWS_PER_TILE, ROWS_PER_TILE), c])


@jax.jit
def _message_passing(x2, src2, dst):
    mesh = plsc.VectorSubcoreMesh(core_axis_name="c", subcore_axis_name="s")
    fn = functools.partial(
        pl.kernel,
        mesh=mesh,
        out_type=jax.ShapeDtypeStruct((N_NODES, 2, HALF), jnp.float32),
        scratch_types=[
            pltpu.VMEM((EDGES_PER_TILE,), jnp.int32),
            pltpu.VMEM((N_CHUNKS, CHUNK), jnp.int32),
            pltpu.VMEM((CHUNK, HALF), jnp.float32),
            pltpu.VMEM((CHUNK, HALF), jnp.float32),
            pltpu.VMEM_SHARED((N_NODES, HALF), jnp.float32),
            pltpu.SemaphoreType.DMA,
            pltpu.SemaphoreType.DMA,
            pltpu.SemaphoreType.DMA,
            pltpu.SemaphoreType.DMA,
        ],
    )(_sc_body)
    return fn(x2, src2, dst)


def kernel(edge_index, x):
    ei = edge_index.astype(jnp.int32)
    src = ei[0].reshape(1, N_SUBCORES, EDGES_PER_TILE)
    src2 = 2 * jnp.concatenate([src, src], axis=0) + jnp.array([0, 1], jnp.int32).reshape(2, 1, 1)
    dst = ei[1].reshape(N_SUBCORES, N_CHUNKS, CHUNK)
    x2 = x.reshape(2 * N_NODES, HALF)
    out = _message_passing(x2, src2, dst)
    return out.reshape(N_NODES, D_FEAT)


# speedup vs baseline: 7.8925x; 1.0026x over previous
import functools

import jax
import jax.numpy as jnp
from jax import lax
from jax.experimental import pallas as pl
from jax.experimental.pallas import tpu as pltpu
from jax.experimental.pallas import tpu_sc as plsc

N_NODES = 10000
D_FEAT = 256
N_EDGES = 160000

N_SUBCORES = 16
HALF = D_FEAT // 2
EDGES_PER_TILE = N_EDGES // N_SUBCORES
CHUNK = 80
N_CHUNKS = EDGES_PER_TILE // CHUNK
ROWS_PER_TILE = N_NODES // N_SUBCORES


def _sc_body(x_hbm, src_hbm, dst_hbm, out_hbm,
             src_v, dst_v, rows_a, rows_b, acc_sh,
             gsem0, gsem1, ssem0, ssem1):
    c = lax.axis_index("c")
    s = lax.axis_index("s")
    gsems = (gsem0, gsem1)
    ssems = (ssem0, ssem1)
    bufs = (rows_a, rows_b)

    def gstart(j, b):
        pltpu.async_copy(x_hbm.at[src_v.at[pl.ds(j * CHUNK, CHUNK)]],
                         bufs[b], gsems[b])

    def gwait(j, b):
        pltpu.make_async_copy(x_hbm.at[src_v.at[pl.ds(j * CHUNK, CHUNK)]],
                              bufs[b], gsems[b]).wait()

    def sstart(j, b):
        pltpu.async_copy(bufs[b], acc_sh.at[dst_v.at[j]], ssems[b], add=True)

    def swait(j, b):
        pltpu.make_async_copy(bufs[b], acc_sh.at[dst_v.at[j]], ssems[b]).wait()

    def tform(j, _=None):
        def t_col(k, _):
            off = j * CHUNK + k * 16
            v = src_v[pl.ds(off, 16)]
            src_v[pl.ds(off, 16)] = v * 2 + c
            return 0
        return lax.fori_loop(0, CHUNK // 16, t_col, 0)

    pltpu.sync_copy(src_hbm.at[s], src_v)
    pltpu.sync_copy(dst_hbm.at[s], dst_v)
    tform(0)
    gstart(0, 0)
    tform(1)
    tform(2)

    def z_row(i, _):
        def z_col(k, _):
            rows_b[i, pl.ds(k * 16, 16)] = jnp.zeros((16,), jnp.float32)
            return 0
        return lax.fori_loop(0, HALF // 16, z_col, 0)
    lax.fori_loop(0, CHUNK, z_row, 0)
    base = s * ROWS_PER_TILE
    for t in range(ROWS_PER_TILE // CHUNK):
        pltpu.sync_copy(rows_b, acc_sh.at[pl.ds(base + t * CHUNK, CHUNK)])
    rem = ROWS_PER_TILE % CHUNK
    if rem:
        pltpu.sync_copy(rows_b.at[pl.ds(0, rem)],
                        acc_sh.at[pl.ds(base + (ROWS_PER_TILE // CHUNK) * CHUNK, rem)])
    plsc.subcore_barrier()

    gstart(1, 1)
    gwait(0, 0)
    sstart(0, 0)

    def pair(i, _):
        j = 2 * i + 1
        swait(j - 1, 0)
        gstart(j + 1, 0)
        tform(j + 2)
        gwait(j, 1)
        sstart(j, 1)
        swait(j, 1)
        gstart(j + 2, 1)
        tform(j + 3)
        gwait(j + 1, 0)
        sstart(j + 1, 0)
        return 0
    lax.fori_loop(0, (N_CHUNKS - 3) // 2, pair, 0)

    j_last = N_CHUNKS - 1
    swait(j_last - 2, 0)
    gstart(j_last, 0)
    gwait(j_last - 1, 1)
    sstart(j_last - 1, 1)
    swait(j_last - 1, 1)
    gwait(j_last, 0)
    sstart(j_last, 0)
    swait(j_last, 0)
    plsc.subcore_barrier()

    pltpu.sync_copy(acc_sh.at[pl.ds(s * ROWS_PER_TILE, ROWS_PER_TILE)],
                    out_hbm.at[pl.ds(s * ROWS_PER_TILE, ROWS_PER_TILE), c])


@jax.jit
def _message_passing(x2, src, dst):
    mesh = plsc.VectorSubcoreMesh(core_axis_name="c", subcore_axis_name="s")
    fn = functools.partial(
        pl.kernel,
        mesh=mesh,
        out_type=jax.ShapeDtypeStruct((N_NODES, 2, HALF), jnp.float32),
        scratch_types=[
            pltpu.VMEM((EDGES_PER_TILE,), jnp.int32),
            pltpu.VMEM((N_CHUNKS, CHUNK), jnp.int32),
            pltpu.VMEM((CHUNK, HALF), jnp.float32),
            pltpu.VMEM((CHUNK, HALF), jnp.float32),
            pltpu.VMEM_SHARED((N_NODES, HALF), jnp.float32),
            pltpu.SemaphoreType.DMA,
            pltpu.SemaphoreType.DMA,
            pltpu.SemaphoreType.DMA,
            pltpu.SemaphoreType.DMA,
        ],
    )(_sc_body)
    return fn(x2, src, dst)


def kernel(edge_index, x):
    ei = edge_index.astype(jnp.int32)
    src = ei[0].reshape(N_SUBCORES, EDGES_PER_TILE)
    dst = ei[1].reshape(N_SUBCORES, N_CHUNKS, CHUNK)
    x2 = x.reshape(2 * N_NODES, HALF)
    out = _message_passing(x2, src, dst)
    return out.reshape(N_NODES, D_FEAT)


# speedup vs baseline: 8.4016x; 1.0645x over previous
import functools

import jax
import jax.numpy as jnp
from jax import lax
from jax.experimental import pallas as pl
from jax.experimental.pallas import tpu as pltpu
from jax.experimental.pallas import tpu_sc as plsc

N_NODES = 10000
D_FEAT = 256
N_EDGES = 160000

N_SUBCORES = 16
HALF = D_FEAT // 2
EDGES_PER_TILE = N_EDGES // N_SUBCORES
CHUNK = 80
N_CHUNKS = EDGES_PER_TILE // CHUNK
ROWS_PER_TILE = N_NODES // N_SUBCORES


def _sc_body(x_hbm, src_hbm, dst_hbm, out_hbm,
             src_v, dst_v, rows_a, rows_b, acc_sh,
             gsem0, gsem1, ssem0, ssem1):
    c = lax.axis_index("c")
    s = lax.axis_index("s")
    gsems = (gsem0, gsem1)
    ssems = (ssem0, ssem1)
    bufs = (rows_a, rows_b)

    half = pl.ds(c * HALF, HALF)

    def gstart(j, b):
        pltpu.async_copy(x_hbm.at[src_v.at[pl.ds(j * CHUNK, CHUNK)], half],
                         bufs[b], gsems[b])

    def gwait(j, b):
        pltpu.make_async_copy(x_hbm.at[src_v.at[pl.ds(j * CHUNK, CHUNK)], half],
                              bufs[b], gsems[b]).wait()

    def sstart(j, b):
        pltpu.async_copy(bufs[b], acc_sh.at[dst_v.at[j]], ssems[b], add=True)

    def swait(j, b):
        pltpu.make_async_copy(bufs[b], acc_sh.at[dst_v.at[j]], ssems[b]).wait()

    pltpu.sync_copy(src_hbm.at[s], src_v)
    pltpu.sync_copy(dst_hbm.at[s], dst_v)
    gstart(0, 0)

    def z_row(i, _):
        def z_col(k, _):
            rows_b[i, pl.ds(k * 16, 16)] = jnp.zeros((16,), jnp.float32)
            return 0
        return lax.fori_loop(0, HALF // 16, z_col, 0)
    lax.fori_loop(0, CHUNK, z_row, 0)
    base = s * ROWS_PER_TILE
    for t in range(ROWS_PER_TILE // CHUNK):
        pltpu.sync_copy(rows_b, acc_sh.at[pl.ds(base + t * CHUNK, CHUNK)])
    rem = ROWS_PER_TILE % CHUNK
    if rem:
        pltpu.sync_copy(rows_b.at[pl.ds(0, rem)],
                        acc_sh.at[pl.ds(base + (ROWS_PER_TILE // CHUNK) * CHUNK, rem)])
    plsc.subcore_barrier()

    gstart(1, 1)
    gwait(0, 0)
    sstart(0, 0)

    def pair(i, _):
        j = 2 * i + 1
        swait(j - 1, 0)
        gstart(j + 1, 0)
        gwait(j, 1)
        sstart(j, 1)
        swait(j, 1)
        gstart(j + 2, 1)
        gwait(j + 1, 0)
        sstart(j + 1, 0)
        return 0
    lax.fori_loop(0, (N_CHUNKS - 3) // 2, pair, 0)

    j_last = N_CHUNKS - 1
    swait(j_last - 2, 0)
    gstart(j_last, 0)
    gwait(j_last - 1, 1)
    sstart(j_last - 1, 1)
    swait(j_last - 1, 1)
    gwait(j_last, 0)
    sstart(j_last, 0)
    swait(j_last, 0)
    plsc.subcore_barrier()

    pltpu.sync_copy(acc_sh.at[pl.ds(s * ROWS_PER_TILE, ROWS_PER_TILE)],
                    out_hbm.at[pl.ds(s * ROWS_PER_TILE, ROWS_PER_TILE), c])


@jax.jit
def _message_passing(x2, src, dst):
    mesh = plsc.VectorSubcoreMesh(core_axis_name="c", subcore_axis_name="s")
    fn = functools.partial(
        pl.kernel,
        mesh=mesh,
        out_type=jax.ShapeDtypeStruct((N_NODES, 2, HALF), jnp.float32),
        scratch_types=[
            pltpu.VMEM((EDGES_PER_TILE,), jnp.int32),
            pltpu.VMEM((N_CHUNKS, CHUNK), jnp.int32),
            pltpu.VMEM((CHUNK, HALF), jnp.float32),
            pltpu.VMEM((CHUNK, HALF), jnp.float32),
            pltpu.VMEM_SHARED((N_NODES, HALF), jnp.float32),
            pltpu.SemaphoreType.DMA,
            pltpu.SemaphoreType.DMA,
            pltpu.SemaphoreType.DMA,
            pltpu.SemaphoreType.DMA,
        ],
    )(_sc_body)
    return fn(x2, src, dst)


def kernel(edge_index, x):
    ei = edge_index.astype(jnp.int32)
    src = ei[0].reshape(N_SUBCORES, EDGES_PER_TILE)
    dst = ei[1].reshape(N_SUBCORES, N_CHUNKS, CHUNK)
    out = _message_passing(x, src, dst)
    return out.reshape(N_NODES, D_FEAT)


# speedup vs baseline: 9.1681x; 1.0912x over previous
import functools

import jax
import jax.numpy as jnp
from jax import lax
from jax.experimental import pallas as pl
from jax.experimental.pallas import tpu as pltpu
from jax.experimental.pallas import tpu_sc as plsc

N_NODES = 10000
D_FEAT = 256
N_EDGES = 160000

N_SUBCORES = 16
HALF = D_FEAT // 2
EDGES_PER_TILE = N_EDGES // N_SUBCORES
CHUNK = 80
N_CHUNKS = EDGES_PER_TILE // CHUNK
ROWS_PER_TILE = N_NODES // N_SUBCORES


def _sc_body(x_hbm, src_hbm, dst_hbm, out_hbm,
             src_v, dst_v, rows_a, rows_b, acc_sh,
             gsem0, gsem1, ssem0, ssem1):
    c = lax.axis_index("c")
    s = lax.axis_index("s")
    gsems = (gsem0, gsem1)
    ssems = (ssem0, ssem1)
    bufs = (rows_a, rows_b)

    half = pl.ds(c * HALF, HALF)

    def gstart(j, b):
        pltpu.async_copy(x_hbm.at[src_v.at[pl.ds(j * CHUNK, CHUNK)], half],
                         bufs[b], gsems[b])

    def gwait(j, b):
        pltpu.make_async_copy(x_hbm.at[src_v.at[pl.ds(j * CHUNK, CHUNK)], half],
                              bufs[b], gsems[b]).wait()

    def sstart(j, b):
        pltpu.async_copy(bufs[b], acc_sh.at[dst_v.at[j]], ssems[b], add=True)

    def swait(j, b):
        pltpu.make_async_copy(bufs[b], acc_sh.at[dst_v.at[j]], ssems[b]).wait()

    pltpu.sync_copy(src_hbm.at[s], src_v)
    pltpu.sync_copy(dst_hbm.at[s], dst_v)
    gstart(0, 0)

    def z_row(i, _):
        def z_col(k, _):
            rows_b[i, pl.ds(k * 16, 16)] = jnp.zeros((16,), jnp.float32)
            return 0
        return lax.fori_loop(0, HALF // 16, z_col, 0)
    lax.fori_loop(0, CHUNK, z_row, 0)
    base = s * ROWS_PER_TILE
    for t in range(ROWS_PER_TILE // CHUNK):
        pltpu.sync_copy(rows_b, acc_sh.at[pl.ds(base + t * CHUNK, CHUNK)])
    rem = ROWS_PER_TILE % CHUNK
    if rem:
        pltpu.sync_copy(rows_b.at[pl.ds(0, rem)],
                        acc_sh.at[pl.ds(base + (ROWS_PER_TILE // CHUNK) * CHUNK, rem)])
    plsc.subcore_barrier()

    gstart(1, 1)
    gwait(0, 0)
    sstart(0, 0)

    def pair(i, _):
        j = 2 * i + 1
        swait(j - 1, 0)
        gstart(j + 1, 0)
        gwait(j, 1)
        sstart(j, 1)
        swait(j, 1)
        gstart(j + 2, 1)
        gwait(j + 1, 0)
        sstart(j + 1, 0)
        return 0
    lax.fori_loop(0, (N_CHUNKS - 3) // 2, pair, 0)

    j_last = N_CHUNKS - 1
    swait(j_last - 2, 0)
    gstart(j_last, 0)
    gwait(j_last - 1, 1)
    sstart(j_last - 1, 1)
    swait(j_last - 1, 1)
    gwait(j_last, 0)
    sstart(j_last, 0)
    swait(j_last, 0)
    plsc.subcore_barrier()

    blk = (N_NODES // N_SUBCORES) // 8 * 8
    pltpu.sync_copy(acc_sh.at[pl.ds(s * blk, blk)],
                    out_hbm.at[pl.ds(s * blk, blk), half])

    @pl.when(s == N_SUBCORES - 1)
    def _copy_tail():
        tail = N_NODES - blk * N_SUBCORES
        pltpu.sync_copy(acc_sh.at[pl.ds(blk * N_SUBCORES, tail)],
                        out_hbm.at[pl.ds(blk * N_SUBCORES, tail), half])


@jax.jit
def _message_passing(x2, src, dst):
    mesh = plsc.VectorSubcoreMesh(core_axis_name="c", subcore_axis_name="s")
    fn = functools.partial(
        pl.kernel,
        mesh=mesh,
        out_type=jax.ShapeDtypeStruct((N_NODES, D_FEAT), jnp.float32),
        scratch_types=[
            pltpu.VMEM((EDGES_PER_TILE,), jnp.int32),
            pltpu.VMEM((N_CHUNKS, CHUNK), jnp.int32),
            pltpu.VMEM((CHUNK, HALF), jnp.float32),
            pltpu.VMEM((CHUNK, HALF), jnp.float32),
            pltpu.VMEM_SHARED((N_NODES, HALF), jnp.float32),
            pltpu.SemaphoreType.DMA,
            pltpu.SemaphoreType.DMA,
            pltpu.SemaphoreType.DMA,
            pltpu.SemaphoreType.DMA,
        ],
    )(_sc_body)
    return fn(x2, src, dst)


def kernel(edge_index, x):
    ei = edge_index.astype(jnp.int32)
    src = ei[0].reshape(N_SUBCORES, EDGES_PER_TILE)
    dst = ei[1].reshape(N_SUBCORES, N_CHUNKS, CHUNK)
    return _message_passing(x, src, dst)


# speedup vs baseline: 9.1695x; 1.0002x over previous
import functools

import jax
import jax.numpy as jnp
from jax import lax
from jax.experimental import pallas as pl
from jax.experimental.pallas import tpu as pltpu
from jax.experimental.pallas import tpu_sc as plsc

N_NODES = 10000
D_FEAT = 256
N_EDGES = 160000

N_SUBCORES = 16
HALF = D_FEAT // 2
EDGES_PER_TILE = N_EDGES // N_SUBCORES
CHUNK = 80
N_CHUNKS = EDGES_PER_TILE // CHUNK
ROWS_PER_TILE = N_NODES // N_SUBCORES


def _sc_body(x_hbm, src_hbm, dst_hbm, out_hbm,
             src_v, dst_v, rows_a, rows_b, acc_sh,
             gsem0, gsem1, ssem0, ssem1):
    c = lax.axis_index("c")
    s = lax.axis_index("s")
    gsems = (gsem0, gsem1)
    ssems = (ssem0, ssem1)
    bufs = (rows_a, rows_b)

    half = pl.ds(c * HALF, HALF)

    def gstart(j, b):
        pltpu.async_copy(x_hbm.at[src_v.at[pl.ds(j * CHUNK, CHUNK)], half],
                         bufs[b], gsems[b])

    def gwait(j, b):
        pltpu.make_async_copy(x_hbm.at[src_v.at[pl.ds(j * CHUNK, CHUNK)], half],
                              bufs[b], gsems[b]).wait()

    def sstart(j, b):
        pltpu.async_copy(bufs[b], acc_sh.at[dst_v.at[j]], ssems[b], add=True)

    def swait(j, b):
        pltpu.make_async_copy(bufs[b], acc_sh.at[dst_v.at[j]], ssems[b]).wait()

    pltpu.sync_copy(src_hbm.at[s], src_v)
    pltpu.sync_copy(dst_hbm.at[s], dst_v)
    gstart(0, 0)

    def z_row(i, _):
        def z_col(k, _):
            rows_b[i, pl.ds(k * 16, 16)] = jnp.zeros((16,), jnp.float32)
            return 0
        return lax.fori_loop(0, HALF // 16, z_col, 0)
    lax.fori_loop(0, CHUNK, z_row, 0)
    base = s * ROWS_PER_TILE
    zdescs = []
    for t in range(ROWS_PER_TILE // CHUNK):
        zdescs.append(pltpu.async_copy(
            rows_b, acc_sh.at[pl.ds(base + t * CHUNK, CHUNK)], ssem1))
    rem = ROWS_PER_TILE % CHUNK
    if rem:
        zdescs.append(pltpu.async_copy(
            rows_b.at[pl.ds(0, rem)],
            acc_sh.at[pl.ds(base + (ROWS_PER_TILE // CHUNK) * CHUNK, rem)],
            ssem1))
    for d in zdescs:
        d.wait()
    plsc.subcore_barrier()

    gstart(1, 1)
    gwait(0, 0)
    sstart(0, 0)

    def pair(i, _):
        j = 2 * i + 1
        swait(j - 1, 0)
        gstart(j + 1, 0)
        gwait(j, 1)
        sstart(j, 1)
        swait(j, 1)
        gstart(j + 2, 1)
        gwait(j + 1, 0)
        sstart(j + 1, 0)
        return 0
    lax.fori_loop(0, (N_CHUNKS - 3) // 2, pair, 0)

    j_last = N_CHUNKS - 1
    swait(j_last - 2, 0)
    gstart(j_last, 0)
    gwait(j_last - 1, 1)
    sstart(j_last - 1, 1)
    swait(j_last - 1, 1)
    gwait(j_last, 0)
    sstart(j_last, 0)
    swait(j_last, 0)
    plsc.subcore_barrier()

    blk = (N_NODES // N_SUBCORES) // 8 * 8
    pltpu.sync_copy(acc_sh.at[pl.ds(s * blk, blk)],
                    out_hbm.at[pl.ds(s * blk, blk), half])

    @pl.when(s == N_SUBCORES - 1)
    def _copy_tail():
        tail = N_NODES - blk * N_SUBCORES
        pltpu.sync_copy(acc_sh.at[pl.ds(blk * N_SUBCORES, tail)],
                        out_hbm.at[pl.ds(blk * N_SUBCORES, tail), half])


@jax.jit
def _message_passing(x2, src, dst):
    mesh = plsc.VectorSubcoreMesh(core_axis_name="c", subcore_axis_name="s")
    fn = functools.partial(
        pl.kernel,
        mesh=mesh,
        out_type=jax.ShapeDtypeStruct((N_NODES, D_FEAT), jnp.float32),
        scratch_types=[
            pltpu.VMEM((EDGES_PER_TILE,), jnp.int32),
            pltpu.VMEM((N_CHUNKS, CHUNK), jnp.int32),
            pltpu.VMEM((CHUNK, HALF), jnp.float32),
            pltpu.VMEM((CHUNK, HALF), jnp.float32),
            pltpu.VMEM_SHARED((N_NODES, HALF), jnp.float32),
            pltpu.SemaphoreType.DMA,
            pltpu.SemaphoreType.DMA,
            pltpu.SemaphoreType.DMA,
            pltpu.SemaphoreType.DMA,
        ],
    )(_sc_body)
    return fn(x2, src, dst)


def kernel(edge_index, x):
    ei = edge_index.astype(jnp.int32)
    src = ei[0].reshape(N_SUBCORES, EDGES_PER_TILE)
    dst = ei[1].reshape(N_SUBCORES, N_CHUNKS, CHUNK)
    return _message_passing(x, src, dst)
